# baseline (speedup 1.0000x reference)
import jax
import jax.numpy as jnp
from jax import lax
from jax.experimental import pallas as pl
from jax.experimental.pallas import tpu as pltpu
from jax.experimental.pallas import tpu_sc as plsc

_B = 16384
_V = 27
_D = 27 * 27
_P = 768
_NC = 2
_NS = 16
_NW = _NC * _NS
_BPW = _B // _NW
_C = 64
_NCH = _BPW // _C
_TR = 256


def _sc_body(x_hbm, w_hbm, out_hbm, idx_v, rows0, rows1, gsem, wsem):
    wid = lax.axis_index("s") * _NC + lax.axis_index("c")
    base = wid * _BPW
    pltpu.sync_copy(x_hbm.at[pl.ds(base, _BPW)], idx_v)

    bufs = (rows0, rows1)
    writes = [None, None]
    for c in range(_NCH):
        b = c % 2
        if writes[b] is not None:
            writes[b].wait()
        pltpu.async_copy(
            w_hbm.at[idx_v.at[pl.ds(c * _C, _C)]], bufs[b], gsem
        ).wait()
        writes[b] = pltpu.async_copy(
            bufs[b], out_hbm.at[pl.ds(base + c * _C, _C)], wsem
        )
    for h in writes:
        h.wait()


def _depad_body(in_ref, out_ref):
    out_ref[...] = in_ref[:, : _D]


@jax.jit
def kernel(x, W):
    x32 = x.astype(jnp.int32)
    w2d = jnp.pad(W.reshape(_V, _D), ((0, 0), (0, _P - _D)))
    mesh = plsc.VectorSubcoreMesh(core_axis_name="c", subcore_axis_name="s")
    padded = pl.kernel(
        _sc_body,
        out_type=jax.ShapeDtypeStruct((_B, _P), jnp.float32),
        mesh=mesh,
        compiler_params=pltpu.CompilerParams(needs_layout_passes=False),
        scratch_types=[
            pltpu.VMEM((_BPW,), jnp.int32),
            pltpu.VMEM((_C, _P), jnp.float32),
            pltpu.VMEM((_C, _P), jnp.float32),
            pltpu.SemaphoreType.DMA,
            pltpu.SemaphoreType.DMA,
        ],
    )(x32, w2d)
    out = pl.pallas_call(
        _depad_body,
        grid=(_B // _TR,),
        in_specs=[pl.BlockSpec((_TR, _P), lambda i: (i, 0))],
        out_specs=pl.BlockSpec((_TR, _D), lambda i: (i, 0)),
        out_shape=jax.ShapeDtypeStruct((_B, _D), jnp.float32),
    )(padded)
    return out.reshape(_B, 27, 27)


# speedup vs baseline: 2.1617x; 2.1617x over previous
import jax
import jax.numpy as jnp
from jax import lax
from jax.experimental import pallas as pl
from jax.experimental.pallas import tpu as pltpu
from jax.experimental.pallas import tpu_sc as plsc

_B = 16384
_V = 27
_D = 27 * 27
_NC = 2
_NS = 16
_NW = _NC * _NS
_BPW = _B // _NW
_K = 4


def _sc_body(x_hbm, w_hbm, out_hbm, table_v, idx_v, sem):
    wid = lax.axis_index("s") * _NC + lax.axis_index("c")
    base = wid * _BPW
    pltpu.sync_copy(w_hbm, table_v)
    pltpu.sync_copy(x_hbm.at[pl.ds(base, _BPW)], idx_v)

    def group_body(g, _):
        vec = idx_v[pl.ds(g * 16, 16)]
        for l in range(16):
            pltpu.async_copy(
                table_v.at[vec[l]], out_hbm.at[base + g * 16 + l], sem
            )

        @pl.when(g >= _K)
        def _drain():
            pltpu.make_async_copy(
                w_hbm.at[pl.ds(0, 16)], table_v.at[pl.ds(0, 16)], sem
            ).wait()

        return _

    lax.fori_loop(0, _BPW // 16, group_body, 0)
    for _u in range(_K):
        pltpu.make_async_copy(
            w_hbm.at[pl.ds(0, 16)], table_v.at[pl.ds(0, 16)], sem
        ).wait()


@jax.jit
def kernel(x, W):
    x32 = x.astype(jnp.int32)
    w2d = W.reshape(_V, _D)
    mesh = plsc.VectorSubcoreMesh(core_axis_name="c", subcore_axis_name="s")
    out = pl.kernel(
        _sc_body,
        out_type=jax.ShapeDtypeStruct((_B, _D), jnp.float32),
        mesh=mesh,
        compiler_params=pltpu.CompilerParams(needs_layout_passes=False),
        scratch_types=[
            pltpu.VMEM((_V, _D), jnp.float32),
            pltpu.VMEM((_BPW,), jnp.int32),
            pltpu.SemaphoreType.DMA,
        ],
    )(x32, w2d)
    return out.reshape(_B, 27, 27)
